# baseline (speedup 1.0000x reference)
import functools

import jax
import jax.numpy as jnp
from jax import lax
from jax.experimental import pallas as pl
from jax.experimental.pallas import tpu as pltpu
from jax.experimental.pallas import tpu_sc as plsc

N = 10000
E1 = 320000
E2 = 160000
D1 = 5000
D2 = 1000
F_IN = 128
H = 128
C = 47

NC = 2
NS = 16
NW = NC * NS
L = 16

CHUNK = 80

R1 = 5120
E1P = 322560
EPT1 = E1P // NW
NCH1 = EPT1 // CHUNK

R2 = 1024
E2P = 163840
EPT2 = E2P // NW
NCH2 = EPT2 // CHUNK


def _sc_segsum(table, src_r, dst_r, zeros_hbm, *, rows, nchunks):
    mesh = plsc.VectorSubcoreMesh(core_axis_name="c", subcore_axis_name="s")
    per_tile = rows // NS

    @functools.partial(
        pl.kernel,
        mesh=mesh,
        compiler_params=pltpu.CompilerParams(needs_layout_passes=False),
        out_type=[
            jax.ShapeDtypeStruct((NC, rows, F_IN), jnp.float32),
            jax.ShapeDtypeStruct((NW, rows), jnp.float32),
        ],
        scratch_types=[
            pltpu.VMEM((nchunks, CHUNK), jnp.int32),
            pltpu.VMEM((nchunks, CHUNK), jnp.int32),
            pltpu.VMEM((CHUNK, F_IN), jnp.float32),
            pltpu.VMEM((CHUNK, F_IN), jnp.float32),
            pltpu.VMEM((rows,), jnp.float32),
            pltpu.VMEM_SHARED((rows, F_IN), jnp.float32),
            pltpu.SemaphoreType.DMA,
            pltpu.SemaphoreType.DMA,
        ],
    )
    def k(table_hbm, src_hbm, dst_hbm, zero_hbm, parts_hbm, degs_hbm,
          src_v, dst_v, rows0_v, rows1_v, deg_v, acc_sh, sem0, sem1):
        c = lax.axis_index("c")
        s = lax.axis_index("s")
        wid = c * NS + s
        pltpu.sync_copy(zero_hbm.at[pl.ds(s * per_tile, per_tile)],
                        acc_sh.at[pl.ds(s * per_tile, per_tile)])
        pltpu.sync_copy(src_hbm.at[wid], src_v)
        pltpu.sync_copy(dst_hbm.at[wid], dst_v)

        def zbody(i, carry):
            deg_v[pl.ds(pl.multiple_of(i * L, L), L)] = jnp.zeros((L,), jnp.float32)
            return carry

        lax.fori_loop(0, rows // L, zbody, 0)
        plsc.subcore_barrier()

        ones = jnp.ones((L,), jnp.float32)

        def hist(j):
            for kk in range(CHUNK // L):
                idx = dst_v[j, pl.ds(kk * L, L)]
                plsc.addupdate_scatter(deg_v, [idx], ones)

        def gwait(buf, sem):
            pltpu.make_async_copy(table_hbm.at[src_v.at[0]], buf, sem).wait()

        pltpu.async_copy(table_hbm.at[src_v.at[0]], rows0_v, sem0)
        hist(0)

        def body(i, carry):
            j = 2 * i
            gwait(rows0_v, sem0)
            pltpu.async_copy(table_hbm.at[src_v.at[j + 1]], rows1_v, sem1)
            hist(j + 1)
            pltpu.sync_copy(rows0_v, acc_sh.at[dst_v.at[j]], add=True)
            gwait(rows1_v, sem1)
            jn = jnp.minimum(j + 2, nchunks - 1)
            pltpu.async_copy(table_hbm.at[src_v.at[jn]], rows0_v, sem0)

            @pl.when(i + 1 < nchunks // 2)
            def _():
                hist(j + 2)

            pltpu.sync_copy(rows1_v, acc_sh.at[dst_v.at[j + 1]], add=True)
            return carry

        lax.fori_loop(0, nchunks // 2, body, 0)
        gwait(rows0_v, sem0)
        pltpu.sync_copy(deg_v, degs_hbm.at[wid])
        plsc.subcore_barrier()
        pltpu.sync_copy(acc_sh.at[pl.ds(s * per_tile, per_tile)],
                        parts_hbm.at[c, pl.ds(s * per_tile, per_tile)])

    return k(table, src_r, dst_r, zeros_hbm)


def _tc_layer1(parts1, degs1, x5, W1_self, W1_neigh, b1, W2_self, b2):
    def body(p_ref, d_ref, x_ref, w1s_ref, w1n_ref, b1_ref, w2s_ref, b2_ref,
             h_ref, oself_ref):
        feats = (p_ref[0] + p_ref[1])[:D1]
        deg = jnp.maximum(jnp.sum(d_ref[...], axis=0), 1.0)[:D1, None]
        agg = feats / deg
        h = x_ref[...] @ w1s_ref[...] + agg @ w1n_ref[...] + b1_ref[...]
        h = jnp.maximum(h, 0.0)
        h_ref[...] = h
        oself_ref[...] = h[:D2] @ w2s_ref[...] + b2_ref[...]

    return pl.pallas_call(
        body,
        out_shape=[
            jax.ShapeDtypeStruct((D1, H), jnp.float32),
            jax.ShapeDtypeStruct((D2, C), jnp.float32),
        ],
    )(parts1, degs1, x5, W1_self, W1_neigh, b1, W2_self, b2)


def _tc_layer2(parts2, degs2, oself, W2_neigh):
    def body(p_ref, d_ref, os_ref, w2n_ref, out_ref):
        feats = (p_ref[0] + p_ref[1])[:D2]
        deg = jnp.maximum(jnp.sum(d_ref[...], axis=0), 1.0)[:D2, None]
        agg = feats / deg
        out_ref[...] = os_ref[...] + agg @ w2n_ref[...]

    return pl.pallas_call(
        body,
        out_shape=jax.ShapeDtypeStruct((D2, C), jnp.float32),
    )(parts2, degs2, oself, W2_neigh)


def kernel(x, src1, dst1, src2, dst2, num_dst1, num_dst2,
           W1_self, W1_neigh, b1, W2_self, W2_neigh, b2):
    npad1 = E1P - E1
    pad_dst1 = D1 + (jnp.arange(npad1, dtype=jnp.int32) % (R1 - D1))
    src1_p = jnp.concatenate([src1, jnp.zeros((npad1,), jnp.int32)])
    dst1_p = jnp.concatenate([dst1, pad_dst1])
    src1_r = src1_p.reshape(NW, NCH1, CHUNK)
    dst1_r = dst1_p.reshape(NW, NCH1, CHUNK)
    npad2 = E2P - E2
    pad_dst2 = D2 + (jnp.arange(npad2, dtype=jnp.int32) % (R2 - D2))
    src2_p = jnp.concatenate([src2, jnp.zeros((npad2,), jnp.int32)])
    dst2_p = jnp.concatenate([dst2, pad_dst2])
    src2_r = src2_p.reshape(NW, NCH2, CHUNK)
    dst2_r = dst2_p.reshape(NW, NCH2, CHUNK)
    zeros1 = jnp.zeros((R1, F_IN), jnp.float32)
    zeros2 = jnp.zeros((R2, F_IN), jnp.float32)
    x5 = x[:D1]
    b1r = b1.reshape(1, H)
    b2r = b2.reshape(1, C)

    parts1, degs1 = _sc_segsum(x, src1_r, dst1_r, zeros1, rows=R1, nchunks=NCH1)
    h, oself = _tc_layer1(parts1, degs1, x5, W1_self, W1_neigh, b1r, W2_self, b2r)
    parts2, degs2 = _sc_segsum(h, src2_r, dst2_r, zeros2, rows=R2, nchunks=NCH2)
    return _tc_layer2(parts2, degs2, oself, W2_neigh)


# speedup vs baseline: 1.0783x; 1.0783x over previous
import functools

import jax
import jax.numpy as jnp
from jax import lax
from jax.experimental import pallas as pl
from jax.experimental.pallas import tpu as pltpu
from jax.experimental.pallas import tpu_sc as plsc

N = 10000
E1 = 320000
E2 = 160000
D1 = 5000
D2 = 1000
F_IN = 128
H = 128
C = 47

NC = 2
NS = 16
NW = NC * NS
L = 16

CHUNK = 80

R1 = 5120
E1P = 320000
EPT1 = E1P // NW
NCH1 = EPT1 // CHUNK

R2 = 1024
E2P = 163840
EPT2 = E2P // NW
NCH2 = EPT2 // CHUNK


def _sc_segsum(table, src_r, dst_r, zeros_hbm, *, rows, nchunks):
    mesh = plsc.VectorSubcoreMesh(core_axis_name="c", subcore_axis_name="s")
    per_tile = rows // NS

    @functools.partial(
        pl.kernel,
        mesh=mesh,
        compiler_params=pltpu.CompilerParams(needs_layout_passes=False),
        out_type=[
            jax.ShapeDtypeStruct((NC, rows, F_IN), jnp.float32),
            jax.ShapeDtypeStruct((NW, rows), jnp.float32),
        ],
        scratch_types=[
            pltpu.VMEM((nchunks, CHUNK), jnp.int32),
            pltpu.VMEM((nchunks, CHUNK), jnp.int32),
            pltpu.VMEM((CHUNK, F_IN), jnp.float32),
            pltpu.VMEM((rows,), jnp.float32),
            pltpu.VMEM_SHARED((rows, F_IN), jnp.float32),
            pltpu.SemaphoreType.DMA,
        ],
    )
    def k(table_hbm, src_hbm, dst_hbm, zero_hbm, parts_hbm, degs_hbm,
          src_v, dst_v, rows0_v, deg_v, acc_sh, sem0):
        c = lax.axis_index("c")
        s = lax.axis_index("s")
        wid = c * NS + s
        pltpu.sync_copy(zero_hbm.at[pl.ds(s * per_tile, per_tile)],
                        acc_sh.at[pl.ds(s * per_tile, per_tile)])
        pltpu.sync_copy(src_hbm.at[wid], src_v)
        pltpu.sync_copy(dst_hbm.at[wid], dst_v)

        def zbody(i, carry):
            deg_v[pl.ds(pl.multiple_of(i * L, L), L)] = jnp.zeros((L,), jnp.float32)
            return carry

        lax.fori_loop(0, rows // L, zbody, 0)
        plsc.subcore_barrier()

        ones = jnp.ones((L,), jnp.float32)

        def hist(j):
            for kk in range(CHUNK // L):
                idx = dst_v[j, pl.ds(kk * L, L)]
                plsc.addupdate_scatter(deg_v, [idx], ones)

        def body(j, carry):
            cp = pltpu.async_copy(table_hbm.at[src_v.at[j]], rows0_v, sem0)
            hist(j)
            cp.wait()
            pltpu.sync_copy(rows0_v, acc_sh.at[dst_v.at[j]], add=True)
            return carry

        lax.fori_loop(0, nchunks, body, 0)
        pltpu.sync_copy(deg_v, degs_hbm.at[wid])
        plsc.subcore_barrier()
        pltpu.sync_copy(acc_sh.at[pl.ds(s * per_tile, per_tile)],
                        parts_hbm.at[c, pl.ds(s * per_tile, per_tile)])

    return k(table, src_r, dst_r, zeros_hbm)


def _tc_layer1(parts1, degs1, x5, W1_self, W1_neigh, b1, W2_self, b2):
    def body(p_ref, d_ref, x_ref, w1s_ref, w1n_ref, b1_ref, w2s_ref, b2_ref,
             h_ref, oself_ref):
        feats = (p_ref[0] + p_ref[1])[:D1]
        deg = jnp.maximum(jnp.sum(d_ref[...], axis=0), 1.0)[:D1, None]
        agg = feats / deg
        h = x_ref[...] @ w1s_ref[...] + agg @ w1n_ref[...] + b1_ref[...]
        h = jnp.maximum(h, 0.0)
        h_ref[...] = h
        oself_ref[...] = h[:D2] @ w2s_ref[...] + b2_ref[...]

    return pl.pallas_call(
        body,
        out_shape=[
            jax.ShapeDtypeStruct((D1, H), jnp.float32),
            jax.ShapeDtypeStruct((D2, C), jnp.float32),
        ],
    )(parts1, degs1, x5, W1_self, W1_neigh, b1, W2_self, b2)


def _tc_layer2(parts2, degs2, oself, W2_neigh):
    def body(p_ref, d_ref, os_ref, w2n_ref, out_ref):
        feats = (p_ref[0] + p_ref[1])[:D2]
        deg = jnp.maximum(jnp.sum(d_ref[...], axis=0), 1.0)[:D2, None]
        agg = feats / deg
        out_ref[...] = os_ref[...] + agg @ w2n_ref[...]

    return pl.pallas_call(
        body,
        out_shape=jax.ShapeDtypeStruct((D2, C), jnp.float32),
    )(parts2, degs2, oself, W2_neigh)


def kernel(x, src1, dst1, src2, dst2, num_dst1, num_dst2,
           W1_self, W1_neigh, b1, W2_self, W2_neigh, b2):
    src1_r = src1.reshape(NW, NCH1, CHUNK)
    dst1_r = dst1.reshape(NW, NCH1, CHUNK)
    npad2 = E2P - E2
    pad_dst2 = D2 + (jnp.arange(npad2, dtype=jnp.int32) % (R2 - D2))
    src2_p = jnp.concatenate([src2, jnp.zeros((npad2,), jnp.int32)])
    dst2_p = jnp.concatenate([dst2, pad_dst2])
    src2_r = src2_p.reshape(NW, NCH2, CHUNK)
    dst2_r = dst2_p.reshape(NW, NCH2, CHUNK)
    zeros1 = jnp.zeros((R1, F_IN), jnp.float32)
    zeros2 = jnp.zeros((R2, F_IN), jnp.float32)
    x5 = x[:D1]
    b1r = b1.reshape(1, H)
    b2r = b2.reshape(1, C)

    parts1, degs1 = _sc_segsum(x, src1_r, dst1_r, zeros1, rows=R1, nchunks=NCH1)
    h, oself = _tc_layer1(parts1, degs1, x5, W1_self, W1_neigh, b1r, W2_self, b2r)
    parts2, degs2 = _sc_segsum(h, src2_r, dst2_r, zeros2, rows=R2, nchunks=NCH2)
    return _tc_layer2(parts2, degs2, oself, W2_neigh)


# speedup vs baseline: 1.6709x; 1.5495x over previous
import functools

import jax
import jax.numpy as jnp
from jax import lax
from jax.experimental import pallas as pl
from jax.experimental.pallas import tpu as pltpu
from jax.experimental.pallas import tpu_sc as plsc

N = 10000
E1 = 320000
E2 = 160000
D1 = 5000
D2 = 1000
F_IN = 128
H = 128
C = 47

NC = 2
NS = 16
NW = NC * NS
L = 16

CHUNK = 80

R1 = 5120
E1P = 320000
EPT1 = E1P // NW
NCH1 = EPT1 // CHUNK

R2 = 1024
E2P = 163840
EPT2 = E2P // NW
NCH2 = EPT2 // CHUNK


def _sc_segsum(table, src_r, dst_r, zeros_hbm, *, rows, nchunks):
    mesh = plsc.VectorSubcoreMesh(core_axis_name="c", subcore_axis_name="s")
    per_tile = rows // NS

    @functools.partial(
        pl.kernel,
        mesh=mesh,
        compiler_params=pltpu.CompilerParams(needs_layout_passes=False),
        out_type=[
            jax.ShapeDtypeStruct((NC, rows, F_IN), jnp.float32),
            jax.ShapeDtypeStruct((NW, rows), jnp.float32),
        ],
        scratch_types=[
            pltpu.VMEM((nchunks, CHUNK), jnp.int32),
            pltpu.VMEM((nchunks, CHUNK), jnp.int32),
            pltpu.VMEM((CHUNK, F_IN), jnp.float32),
            pltpu.VMEM((rows,), jnp.float32),
            pltpu.VMEM_SHARED((rows, F_IN), jnp.float32),
            pltpu.SemaphoreType.DMA,
        ],
    )
    def k(table_hbm, src_hbm, dst_hbm, zero_hbm, parts_hbm, degs_hbm,
          src_v, dst_v, rows0_v, deg_v, acc_sh, sem0):
        c = lax.axis_index("c")
        s = lax.axis_index("s")
        wid = c * NS + s
        pltpu.sync_copy(zero_hbm.at[pl.ds(s * per_tile, per_tile)],
                        acc_sh.at[pl.ds(s * per_tile, per_tile)])
        pltpu.sync_copy(src_hbm.at[wid], src_v)
        pltpu.sync_copy(dst_hbm.at[wid], dst_v)

        def zbody(i, carry):
            deg_v[pl.ds(pl.multiple_of(i * L, L), L)] = jnp.zeros((L,), jnp.float32)
            return carry

        lax.fori_loop(0, rows // L, zbody, 0)
        plsc.subcore_barrier()

        ones = jnp.ones((L,), jnp.float32)

        def hist(j):
            for kk in range(CHUNK // L):
                idx = dst_v[j, pl.ds(kk * L, L)]
                plsc.addupdate_scatter(deg_v, [idx], ones)

        def body(j, carry):
            cp = pltpu.async_copy(table_hbm.at[src_v.at[j]], rows0_v, sem0)
            hist(j)
            cp.wait()
            pltpu.sync_copy(rows0_v, acc_sh.at[dst_v.at[j]], add=True)
            return carry

        lax.fori_loop(0, nchunks, body, 0)
        pltpu.sync_copy(deg_v, degs_hbm.at[wid])
        plsc.subcore_barrier()
        pltpu.sync_copy(acc_sh.at[pl.ds(s * per_tile, per_tile)],
                        parts_hbm.at[c, pl.ds(s * per_tile, per_tile)])

    return k(table, src_r, dst_r, zeros_hbm)


def _tc_layer1(parts1, degs1, x5, W1_self, W1_neigh, b1, W2_self, b2):
    def body(p_ref, d_ref, x_ref, w1s_ref, w1n_ref, b1_ref, w2s_ref, b2_ref,
             h_ref, oself_ref):
        feats = (p_ref[0] + p_ref[1])[:D1]
        deg = jnp.maximum(jnp.sum(d_ref[...], axis=0), 1.0)[:D1, None]
        agg = feats / deg
        h = x_ref[...] @ w1s_ref[...] + agg @ w1n_ref[...] + b1_ref[...]
        h = jnp.maximum(h, 0.0)
        h_ref[...] = h
        oself_ref[...] = h[:D2] @ w2s_ref[...] + b2_ref[...]

    return pl.pallas_call(
        body,
        out_shape=[
            jax.ShapeDtypeStruct((D1, H), jnp.float32),
            jax.ShapeDtypeStruct((D2, C), jnp.float32),
        ],
    )(parts1, degs1, x5, W1_self, W1_neigh, b1, W2_self, b2)


def _tc_layer2(parts2, degs2, oself, W2_neigh):
    def body(p_ref, d_ref, os_ref, w2n_ref, out_ref):
        feats = (p_ref[0] + p_ref[1])[:D2]
        deg = jnp.maximum(jnp.sum(d_ref[...], axis=0), 1.0)[:D2, None]
        agg = feats / deg
        out_ref[...] = os_ref[...] + agg @ w2n_ref[...]

    return pl.pallas_call(
        body,
        out_shape=jax.ShapeDtypeStruct((D2, C), jnp.float32),
    )(parts2, degs2, oself, W2_neigh)


def kernel(x, src1, dst1, src2, dst2, num_dst1, num_dst2,
           W1_self, W1_neigh, b1, W2_self, W2_neigh, b2):
    src1_r = src1.reshape(NW, NCH1, CHUNK)
    dst1_r = dst1.reshape(NW, NCH1, CHUNK)
    npad2 = E2P - E2
    pad_idx2 = jnp.arange(npad2, dtype=jnp.int32)
    src2_p = jnp.concatenate([src2, pad_idx2 % D2])
    dst2_p = jnp.concatenate([dst2, D2 + pad_idx2 % (R2 - D2)])
    src2_r = src2_p.reshape(NW, NCH2, CHUNK)
    dst2_r = dst2_p.reshape(NW, NCH2, CHUNK)
    zeros1 = jnp.zeros((R1, F_IN), jnp.float32)
    zeros2 = jnp.zeros((R2, F_IN), jnp.float32)
    x5 = x[:D1]
    b1r = b1.reshape(1, H)
    b2r = b2.reshape(1, C)

    parts1, degs1 = _sc_segsum(x, src1_r, dst1_r, zeros1, rows=R1, nchunks=NCH1)
    h, oself = _tc_layer1(parts1, degs1, x5, W1_self, W1_neigh, b1r, W2_self, b2r)
    parts2, degs2 = _sc_segsum(h, src2_r, dst2_r, zeros2, rows=R2, nchunks=NCH2)
    return _tc_layer2(parts2, degs2, oself, W2_neigh)


# speedup vs baseline: 1.9423x; 1.1624x over previous
import functools

import jax
import jax.numpy as jnp
from jax import lax
from jax.experimental import pallas as pl
from jax.experimental.pallas import tpu as pltpu
from jax.experimental.pallas import tpu_sc as plsc

N = 10000
E1 = 320000
E2 = 160000
D1 = 5000
D2 = 1000
F_IN = 128
H = 128
C = 47

NC = 2
NS = 16
NW = NC * NS
L = 16

CHUNK = 128

R1 = 5120
E1P = 327680
EPT1 = E1P // NW
NCH1 = EPT1 // CHUNK

R2 = 1024
E2P = 163840
EPT2 = E2P // NW
NCH2 = EPT2 // CHUNK


def _sc_segsum(table, src_r, dst_r, zeros_hbm, *, rows, nchunks):
    mesh = plsc.VectorSubcoreMesh(core_axis_name="c", subcore_axis_name="s")
    per_tile = rows // NS

    @functools.partial(
        pl.kernel,
        mesh=mesh,
        compiler_params=pltpu.CompilerParams(needs_layout_passes=False),
        out_type=[
            jax.ShapeDtypeStruct((NC, rows, F_IN), jnp.float32),
            jax.ShapeDtypeStruct((NW, rows), jnp.float32),
        ],
        scratch_types=[
            pltpu.VMEM((nchunks, CHUNK), jnp.int32),
            pltpu.VMEM((nchunks, CHUNK), jnp.int32),
            pltpu.VMEM((CHUNK, F_IN), jnp.float32),
            pltpu.VMEM((rows,), jnp.float32),
            pltpu.VMEM_SHARED((rows, F_IN), jnp.float32),
            pltpu.SemaphoreType.DMA,
        ],
    )
    def k(table_hbm, src_hbm, dst_hbm, zero_hbm, parts_hbm, degs_hbm,
          src_v, dst_v, rows0_v, deg_v, acc_sh, sem0):
        c = lax.axis_index("c")
        s = lax.axis_index("s")
        wid = c * NS + s
        pltpu.sync_copy(zero_hbm.at[pl.ds(s * per_tile, per_tile)],
                        acc_sh.at[pl.ds(s * per_tile, per_tile)])
        pltpu.sync_copy(src_hbm.at[wid], src_v)
        pltpu.sync_copy(dst_hbm.at[wid], dst_v)

        def zbody(i, carry):
            deg_v[pl.ds(pl.multiple_of(i * L, L), L)] = jnp.zeros((L,), jnp.float32)
            return carry

        lax.fori_loop(0, rows // L, zbody, 0)
        plsc.subcore_barrier()

        ones = jnp.ones((L,), jnp.float32)

        def hist(j):
            for kk in range(CHUNK // L):
                idx = dst_v[j, pl.ds(kk * L, L)]
                plsc.addupdate_scatter(deg_v, [idx], ones)

        def body(j, carry):
            cp = pltpu.async_copy(table_hbm.at[src_v.at[j]], rows0_v, sem0)
            hist(j)
            cp.wait()
            pltpu.sync_copy(rows0_v, acc_sh.at[dst_v.at[j]], add=True)
            return carry

        lax.fori_loop(0, nchunks, body, 0)
        pltpu.sync_copy(deg_v, degs_hbm.at[wid])
        plsc.subcore_barrier()
        pltpu.sync_copy(acc_sh.at[pl.ds(s * per_tile, per_tile)],
                        parts_hbm.at[c, pl.ds(s * per_tile, per_tile)])

    return k(table, src_r, dst_r, zeros_hbm)


def _tc_layer1(parts1, degs1, x5, W1_self, W1_neigh, b1, W2_self, b2):
    def body(p_ref, d_ref, x_ref, w1s_ref, w1n_ref, b1_ref, w2s_ref, b2_ref,
             h_ref, oself_ref):
        feats = (p_ref[0] + p_ref[1])[:D1]
        deg = jnp.maximum(jnp.sum(d_ref[...], axis=0), 1.0)[:D1, None]
        agg = feats / deg
        h = x_ref[...] @ w1s_ref[...] + agg @ w1n_ref[...] + b1_ref[...]
        h = jnp.maximum(h, 0.0)
        h_ref[...] = h
        oself_ref[...] = h[:D2] @ w2s_ref[...] + b2_ref[...]

    return pl.pallas_call(
        body,
        out_shape=[
            jax.ShapeDtypeStruct((D1, H), jnp.float32),
            jax.ShapeDtypeStruct((D2, C), jnp.float32),
        ],
    )(parts1, degs1, x5, W1_self, W1_neigh, b1, W2_self, b2)


def _tc_layer2(parts2, degs2, oself, W2_neigh):
    def body(p_ref, d_ref, os_ref, w2n_ref, out_ref):
        feats = (p_ref[0] + p_ref[1])[:D2]
        deg = jnp.maximum(jnp.sum(d_ref[...], axis=0), 1.0)[:D2, None]
        agg = feats / deg
        out_ref[...] = os_ref[...] + agg @ w2n_ref[...]

    return pl.pallas_call(
        body,
        out_shape=jax.ShapeDtypeStruct((D2, C), jnp.float32),
    )(parts2, degs2, oself, W2_neigh)


def kernel(x, src1, dst1, src2, dst2, num_dst1, num_dst2,
           W1_self, W1_neigh, b1, W2_self, W2_neigh, b2):
    npad1 = E1P - E1
    pad_idx1 = jnp.arange(npad1, dtype=jnp.int32)
    src1_p = jnp.concatenate([src1, pad_idx1 % N])
    dst1_p = jnp.concatenate([dst1, D1 + pad_idx1 % (R1 - D1)])
    src1_r = src1_p.reshape(NW, NCH1, CHUNK)
    dst1_r = dst1_p.reshape(NW, NCH1, CHUNK)
    npad2 = E2P - E2
    pad_idx2 = jnp.arange(npad2, dtype=jnp.int32)
    src2_p = jnp.concatenate([src2, pad_idx2 % D2])
    dst2_p = jnp.concatenate([dst2, D2 + pad_idx2 % (R2 - D2)])
    src2_r = src2_p.reshape(NW, NCH2, CHUNK)
    dst2_r = dst2_p.reshape(NW, NCH2, CHUNK)
    zeros1 = jnp.zeros((R1, F_IN), jnp.float32)
    zeros2 = jnp.zeros((R2, F_IN), jnp.float32)
    x5 = x[:D1]
    b1r = b1.reshape(1, H)
    b2r = b2.reshape(1, C)

    parts1, degs1 = _sc_segsum(x, src1_r, dst1_r, zeros1, rows=R1, nchunks=NCH1)
    h, oself = _tc_layer1(parts1, degs1, x5, W1_self, W1_neigh, b1r, W2_self, b2r)
    parts2, degs2 = _sc_segsum(h, src2_r, dst2_r, zeros2, rows=R2, nchunks=NCH2)
    return _tc_layer2(parts2, degs2, oself, W2_neigh)


# speedup vs baseline: 2.5401x; 1.3078x over previous
import functools

import jax
import jax.numpy as jnp
from jax import lax
from jax.experimental import pallas as pl
from jax.experimental.pallas import tpu as pltpu
from jax.experimental.pallas import tpu_sc as plsc

N = 10000
E1 = 320000
E2 = 160000
D1 = 5000
D2 = 1000
F_IN = 128
H = 128
C = 47

NC = 2
NS = 16
NW = NC * NS
L = 16

CHUNK = 128

R1 = 5120
E1P = 327680
EPT1 = E1P // NW
NCH1 = EPT1 // CHUNK

R2 = 1024
E2P = 163840
EPT2 = E2P // NW
NCH2 = EPT2 // CHUNK


def _sc_segsum(table, src_r, dst_r, zeros_hbm, *, rows, nchunks):
    mesh = plsc.VectorSubcoreMesh(core_axis_name="c", subcore_axis_name="s")
    per_tile = rows // NS

    @functools.partial(
        pl.kernel,
        mesh=mesh,
        compiler_params=pltpu.CompilerParams(needs_layout_passes=False),
        out_type=[
            jax.ShapeDtypeStruct((NC, rows, F_IN), jnp.float32),
            jax.ShapeDtypeStruct((NW, rows), jnp.float32),
        ],
        scratch_types=[
            pltpu.VMEM((nchunks, CHUNK), jnp.int32),
            pltpu.VMEM((nchunks, CHUNK), jnp.int32),
            pltpu.VMEM((CHUNK, F_IN), jnp.float32),
            pltpu.VMEM((CHUNK, F_IN), jnp.float32),
            pltpu.VMEM((rows,), jnp.float32),
            pltpu.VMEM_SHARED((rows, F_IN), jnp.float32),
            pltpu.SemaphoreType.DMA,
            pltpu.SemaphoreType.DMA,
        ],
    )
    def k(table_hbm, src_hbm, dst_hbm, zero_hbm, parts_hbm, degs_hbm,
          src_v, dst_v, rows0_v, rows1_v, deg_v, acc_sh, sem0, sem1):
        c = lax.axis_index("c")
        s = lax.axis_index("s")
        wid = c * NS + s
        pltpu.sync_copy(zero_hbm.at[pl.ds(s * per_tile, per_tile)],
                        acc_sh.at[pl.ds(s * per_tile, per_tile)])
        pltpu.sync_copy(src_hbm.at[wid], src_v)
        pltpu.sync_copy(dst_hbm.at[wid], dst_v)

        def zbody(i, carry):
            deg_v[pl.ds(pl.multiple_of(i * L, L), L)] = jnp.zeros((L,), jnp.float32)
            return carry

        lax.fori_loop(0, rows // L, zbody, 0)
        plsc.subcore_barrier()

        ones = jnp.ones((L,), jnp.float32)

        def hist(j):
            for kk in range(CHUNK // L):
                idx = dst_v[j, pl.ds(kk * L, L)]
                plsc.addupdate_scatter(deg_v, [idx], ones)

        def gwait(buf, sem):
            pltpu.make_async_copy(table_hbm.at[src_v.at[0]], buf, sem).wait()

        pltpu.async_copy(table_hbm.at[src_v.at[0]], rows0_v, sem0)
        hist(0)

        def body(i, carry):
            j = 2 * i
            gwait(rows0_v, sem0)
            pltpu.async_copy(table_hbm.at[src_v.at[j + 1]], rows1_v, sem1)
            hist(j + 1)
            pltpu.sync_copy(rows0_v, acc_sh.at[dst_v.at[j]], add=True)
            gwait(rows1_v, sem1)
            jn = jnp.minimum(j + 2, nchunks - 1)
            pltpu.async_copy(table_hbm.at[src_v.at[jn]], rows0_v, sem0)

            @pl.when(i + 1 < nchunks // 2)
            def _():
                hist(j + 2)

            pltpu.sync_copy(rows1_v, acc_sh.at[dst_v.at[j + 1]], add=True)
            return carry

        lax.fori_loop(0, nchunks // 2, body, 0)
        gwait(rows0_v, sem0)
        pltpu.sync_copy(deg_v, degs_hbm.at[wid])
        plsc.subcore_barrier()
        pltpu.sync_copy(acc_sh.at[pl.ds(s * per_tile, per_tile)],
                        parts_hbm.at[c, pl.ds(s * per_tile, per_tile)])

    return k(table, src_r, dst_r, zeros_hbm)


def _tc_layer1(parts1, degs1, x5, W1_self, W1_neigh, b1, W2_self, b2):
    def body(p_ref, d_ref, x_ref, w1s_ref, w1n_ref, b1_ref, w2s_ref, b2_ref,
             h_ref, oself_ref):
        feats = (p_ref[0] + p_ref[1])[:D1]
        deg = jnp.maximum(jnp.sum(d_ref[...], axis=0), 1.0)[:D1, None]
        agg = feats / deg
        h = x_ref[...] @ w1s_ref[...] + agg @ w1n_ref[...] + b1_ref[...]
        h = jnp.maximum(h, 0.0)
        h_ref[...] = h
        oself_ref[...] = h[:D2] @ w2s_ref[...] + b2_ref[...]

    return pl.pallas_call(
        body,
        out_shape=[
            jax.ShapeDtypeStruct((D1, H), jnp.float32),
            jax.ShapeDtypeStruct((D2, C), jnp.float32),
        ],
    )(parts1, degs1, x5, W1_self, W1_neigh, b1, W2_self, b2)


def _tc_layer2(parts2, degs2, oself, W2_neigh):
    def body(p_ref, d_ref, os_ref, w2n_ref, out_ref):
        feats = (p_ref[0] + p_ref[1])[:D2]
        deg = jnp.maximum(jnp.sum(d_ref[...], axis=0), 1.0)[:D2, None]
        agg = feats / deg
        out_ref[...] = os_ref[...] + agg @ w2n_ref[...]

    return pl.pallas_call(
        body,
        out_shape=jax.ShapeDtypeStruct((D2, C), jnp.float32),
    )(parts2, degs2, oself, W2_neigh)


def kernel(x, src1, dst1, src2, dst2, num_dst1, num_dst2,
           W1_self, W1_neigh, b1, W2_self, W2_neigh, b2):
    npad1 = E1P - E1
    pad_idx1 = jnp.arange(npad1, dtype=jnp.int32)
    src1_p = jnp.concatenate([src1, pad_idx1 % N])
    dst1_p = jnp.concatenate([dst1, D1 + pad_idx1 % (R1 - D1)])
    src1_r = src1_p.reshape(NW, NCH1, CHUNK)
    dst1_r = dst1_p.reshape(NW, NCH1, CHUNK)
    npad2 = E2P - E2
    pad_idx2 = jnp.arange(npad2, dtype=jnp.int32)
    src2_p = jnp.concatenate([src2, pad_idx2 % D2])
    dst2_p = jnp.concatenate([dst2, D2 + pad_idx2 % (R2 - D2)])
    src2_r = src2_p.reshape(NW, NCH2, CHUNK)
    dst2_r = dst2_p.reshape(NW, NCH2, CHUNK)
    zeros1 = jnp.zeros((R1, F_IN), jnp.float32)
    zeros2 = jnp.zeros((R2, F_IN), jnp.float32)
    x5 = x[:D1]
    b1r = b1.reshape(1, H)
    b2r = b2.reshape(1, C)

    parts1, degs1 = _sc_segsum(x, src1_r, dst1_r, zeros1, rows=R1, nchunks=NCH1)
    h, oself = _tc_layer1(parts1, degs1, x5, W1_self, W1_neigh, b1r, W2_self, b2r)
    parts2, degs2 = _sc_segsum(h, src2_r, dst2_r, zeros2, rows=R2, nchunks=NCH2)
    return _tc_layer2(parts2, degs2, oself, W2_neigh)


# speedup vs baseline: 2.5796x; 1.0155x over previous
import functools

import jax
import jax.numpy as jnp
from jax import lax
from jax.experimental import pallas as pl
from jax.experimental.pallas import tpu as pltpu
from jax.experimental.pallas import tpu_sc as plsc

N = 10000
E1 = 320000
E2 = 160000
D1 = 5000
D2 = 1000
F_IN = 128
H = 128
C = 47

NC = 2
NS = 16
NW = NC * NS
L = 16

CHUNK = 128

R1 = 5120
E1P = 327680
EPT1 = E1P // NW
NCH1 = EPT1 // CHUNK

R2 = 1024
E2P = 163840
EPT2 = E2P // NW
NCH2 = EPT2 // CHUNK


def _sc_segsum(table, src_r, dst_r, *, rows, nchunks):
    mesh = plsc.VectorSubcoreMesh(core_axis_name="c", subcore_axis_name="s")
    per_tile = rows // NS

    @functools.partial(
        pl.kernel,
        mesh=mesh,
        compiler_params=pltpu.CompilerParams(needs_layout_passes=False),
        out_type=[
            jax.ShapeDtypeStruct((NC, rows, F_IN), jnp.float32),
            jax.ShapeDtypeStruct((NW, rows), jnp.float32),
        ],
        scratch_types=[
            pltpu.VMEM((nchunks, CHUNK), jnp.int32),
            pltpu.VMEM((nchunks, CHUNK), jnp.int32),
            pltpu.VMEM((CHUNK, F_IN), jnp.float32),
            pltpu.VMEM((CHUNK, F_IN), jnp.float32),
            pltpu.VMEM((rows,), jnp.float32),
            pltpu.VMEM_SHARED((rows, F_IN), jnp.float32),
            pltpu.SemaphoreType.DMA,
            pltpu.SemaphoreType.DMA,
        ],
    )
    def k(table_hbm, src_hbm, dst_hbm, parts_hbm, degs_hbm,
          src_v, dst_v, rows0_v, rows1_v, deg_v, acc_sh, sem0, sem1):
        c = lax.axis_index("c")
        s = lax.axis_index("s")
        wid = c * NS + s
        pltpu.sync_copy(src_hbm.at[wid], src_v)
        pltpu.sync_copy(dst_hbm.at[wid], dst_v)

        def zrow(i, carry):
            for kk in range(F_IN // L):
                rows0_v[i, pl.ds(kk * L, L)] = jnp.zeros((L,), jnp.float32)
            return carry

        lax.fori_loop(0, CHUNK, zrow, 0)
        off = 0
        while off < per_tile:
            size = min(CHUNK, per_tile - off)
            pltpu.sync_copy(rows0_v.at[pl.ds(0, size)],
                            acc_sh.at[pl.ds(s * per_tile + off, size)])
            off += size

        def zbody(i, carry):
            deg_v[pl.ds(pl.multiple_of(i * L, L), L)] = jnp.zeros((L,), jnp.float32)
            return carry

        lax.fori_loop(0, rows // L, zbody, 0)
        plsc.subcore_barrier()

        ones = jnp.ones((L,), jnp.float32)

        def hist(j):
            for kk in range(CHUNK // L):
                idx = dst_v[j, pl.ds(kk * L, L)]
                plsc.addupdate_scatter(deg_v, [idx], ones)

        def gwait(buf, sem):
            pltpu.make_async_copy(table_hbm.at[src_v.at[0]], buf, sem).wait()

        pltpu.async_copy(table_hbm.at[src_v.at[0]], rows0_v, sem0)
        hist(0)

        def body(i, carry):
            j = 2 * i
            gwait(rows0_v, sem0)
            pltpu.async_copy(table_hbm.at[src_v.at[j + 1]], rows1_v, sem1)
            hist(j + 1)
            pltpu.sync_copy(rows0_v, acc_sh.at[dst_v.at[j]], add=True)
            gwait(rows1_v, sem1)
            jn = jnp.minimum(j + 2, nchunks - 1)
            pltpu.async_copy(table_hbm.at[src_v.at[jn]], rows0_v, sem0)

            @pl.when(i + 1 < nchunks // 2)
            def _():
                hist(j + 2)

            pltpu.sync_copy(rows1_v, acc_sh.at[dst_v.at[j + 1]], add=True)
            return carry

        lax.fori_loop(0, nchunks // 2, body, 0)
        gwait(rows0_v, sem0)
        pltpu.sync_copy(deg_v, degs_hbm.at[wid])
        plsc.subcore_barrier()
        pltpu.sync_copy(acc_sh.at[pl.ds(s * per_tile, per_tile)],
                        parts_hbm.at[c, pl.ds(s * per_tile, per_tile)])

    return k(table, src_r, dst_r)


def _tc_layer1(parts1, degs1, x5, W1_self, W1_neigh, b1, W2_self, b2):
    def body(p_ref, d_ref, x_ref, w1s_ref, w1n_ref, b1_ref, w2s_ref, b2_ref,
             h_ref, oself_ref):
        feats = (p_ref[0] + p_ref[1])[:D1]
        deg = jnp.maximum(jnp.sum(d_ref[...], axis=0), 1.0)[:D1, None]
        agg = feats / deg
        h = x_ref[...] @ w1s_ref[...] + agg @ w1n_ref[...] + b1_ref[...]
        h = jnp.maximum(h, 0.0)
        h_ref[...] = h
        oself_ref[...] = h[:D2] @ w2s_ref[...] + b2_ref[...]

    return pl.pallas_call(
        body,
        out_shape=[
            jax.ShapeDtypeStruct((D1, H), jnp.float32),
            jax.ShapeDtypeStruct((D2, C), jnp.float32),
        ],
    )(parts1, degs1, x5, W1_self, W1_neigh, b1, W2_self, b2)


def _tc_layer2(parts2, degs2, oself, W2_neigh):
    def body(p_ref, d_ref, os_ref, w2n_ref, out_ref):
        feats = (p_ref[0] + p_ref[1])[:D2]
        deg = jnp.maximum(jnp.sum(d_ref[...], axis=0), 1.0)[:D2, None]
        agg = feats / deg
        out_ref[...] = os_ref[...] + agg @ w2n_ref[...]

    return pl.pallas_call(
        body,
        out_shape=jax.ShapeDtypeStruct((D2, C), jnp.float32),
    )(parts2, degs2, oself, W2_neigh)


def kernel(x, src1, dst1, src2, dst2, num_dst1, num_dst2,
           W1_self, W1_neigh, b1, W2_self, W2_neigh, b2):
    npad1 = E1P - E1
    pad_idx1 = jnp.arange(npad1, dtype=jnp.int32)
    src1_p = jnp.concatenate([src1, pad_idx1 % N])
    dst1_p = jnp.concatenate([dst1, D1 + pad_idx1 % (R1 - D1)])
    src1_r = src1_p.reshape(NW, NCH1, CHUNK)
    dst1_r = dst1_p.reshape(NW, NCH1, CHUNK)
    npad2 = E2P - E2
    pad_idx2 = jnp.arange(npad2, dtype=jnp.int32)
    src2_p = jnp.concatenate([src2, pad_idx2 % D2])
    dst2_p = jnp.concatenate([dst2, D2 + pad_idx2 % (R2 - D2)])
    src2_r = src2_p.reshape(NW, NCH2, CHUNK)
    dst2_r = dst2_p.reshape(NW, NCH2, CHUNK)
    b1r = b1.reshape(1, H)
    b2r = b2.reshape(1, C)
    x5 = x[:D1]

    parts1, degs1 = _sc_segsum(x, src1_r, dst1_r, rows=R1, nchunks=NCH1)
    h, oself = _tc_layer1(parts1, degs1, x5, W1_self, W1_neigh, b1r, W2_self, b2r)
    parts2, degs2 = _sc_segsum(h, src2_r, dst2_r, rows=R2, nchunks=NCH2)
    return _tc_layer2(parts2, degs2, oself, W2_neigh)
